# baseline (speedup 1.0000x reference)
import functools

import jax
import jax.numpy as jnp
from jax.experimental import pallas as pl
from jax.experimental.pallas import tpu as pltpu

B, N, D = 64, 400, 128
EPS = 1e-5
K_TILES = 8


def _gcn_body(sc_ref, W1_ref, b1_ref, W2_ref, b2_ref, W3_ref, b3_ref,
              g1_ref, be1_ref, g2_ref, be2_ref, g3_ref, be3_ref, out_ref):
    sc = sc_ref[0]
    row = jax.lax.broadcasted_iota(jnp.int32, (N, N), 0)
    col = jax.lax.broadcasted_iota(jnp.int32, (N, N), 1)
    A = sc + jnp.where((row == col) & (sc == 0.0), 1.0, 0.0)
    deg_c = jnp.sum(A, axis=0, keepdims=True)
    ones = jnp.ones((N, 1), jnp.float32)
    deg_r = jax.lax.dot_general(A, ones, (((0,), (0,)), ((), ())),
                                preferred_element_type=jnp.float32, precision=jax.lax.Precision.HIGHEST)
    dinv_c = jnp.where(deg_c > 0, jax.lax.rsqrt(deg_c), 0.0)
    dinv_r = jnp.where(deg_r > 0, jax.lax.rsqrt(deg_r), 0.0)
    Ms = A * dinv_r * dinv_c

    def prop(y):
        return jax.lax.dot_general(Ms, y, (((0,), (0,)), ((), ())),
                                   preferred_element_type=jnp.float32, precision=jax.lax.Precision.HIGHEST)

    def feat(x, w_ref):
        return jax.lax.dot_general(x, w_ref[...], (((1,), (1,)), ((), ())),
                                   preferred_element_type=jnp.float32, precision=jax.lax.Precision.HIGHEST)

    def bn(x, g_ref, b_ref):
        mu = jnp.mean(x, axis=0, keepdims=True)
        var = jnp.mean((x - mu) ** 2, axis=0, keepdims=True)
        return (x - mu) * jax.lax.rsqrt(var + EPS) * g_ref[...] + b_ref[...]

    x = jax.nn.relu(bn(prop(feat(sc, W1_ref)) + b1_ref[...], g1_ref, be1_ref))
    x = jax.nn.relu(bn(prop(feat(x, W2_ref)) + b2_ref[...], g2_ref, be2_ref))
    x = bn(prop(feat(x, W3_ref)) + b3_ref[...], g3_ref, be3_ref)
    out_ref[0] = x


def _clf_body(xf_ref, cW1_ref, cb1_ref, lg1_ref, lb1_ref,
              cW2_ref, cb2_ref, lg2_ref, lb2_ref, cW3_ref, cb3_ref,
              out_ref, acc_ref):
    k = pl.program_id(0)

    @pl.when(k == 0)
    def _init():
        acc_ref[...] = jnp.zeros_like(acc_ref)

    acc_ref[...] += jax.lax.dot_general(
        xf_ref[...], cW1_ref[...], (((1,), (1,)), ((), ())),
        preferred_element_type=jnp.float32, precision=jax.lax.Precision.HIGHEST)

    @pl.when(k == pl.num_programs(0) - 1)
    def _finish():
        def ln(x, g_ref, b_ref):
            mu = jnp.mean(x, axis=-1, keepdims=True)
            var = jnp.mean((x - mu) ** 2, axis=-1, keepdims=True)
            return (x - mu) * jax.lax.rsqrt(var + EPS) * g_ref[...] + b_ref[...]

        h = acc_ref[...] + cb1_ref[...]
        h = jax.nn.relu(ln(h, lg1_ref, lb1_ref))
        h = jax.lax.dot_general(h, cW2_ref[...], (((1,), (1,)), ((), ())),
                                preferred_element_type=jnp.float32, precision=jax.lax.Precision.HIGHEST) + cb2_ref[...]
        h = jax.nn.relu(ln(h, lg2_ref, lb2_ref))
        out_ref[...] = jax.lax.dot_general(
            h, cW3_ref[...], (((1,), (1,)), ((), ())),
            preferred_element_type=jnp.float32, precision=jax.lax.Precision.HIGHEST) + cb3_ref[...]


def _full(spec_shape):
    nd = len(spec_shape)
    return pl.BlockSpec(spec_shape, lambda *_: (0,) * nd)


@functools.partial(jax.jit, static_argnames=("interpret",))
def kernel(fc_matrix, sc_matrix, W1, b1, W2, b2, W3, b3,
           g1, be1, g2, be2, g3, be3,
           cW1, cb1, lg1, lb1, cW2, cb2, lg2, lb2, cW3, cb3,
           interpret=False):
    del fc_matrix
    x = pl.pallas_call(
        _gcn_body,
        grid=(B,),
        in_specs=[
            pl.BlockSpec((1, N, N), lambda b: (b, 0, 0)),
            _full((D, N)), _full((D,)),
            _full((D, D)), _full((D,)),
            _full((D, D)), _full((D,)),
            _full((D,)), _full((D,)),
            _full((D,)), _full((D,)),
            _full((D,)), _full((D,)),
        ],
        out_specs=pl.BlockSpec((1, N, D), lambda b: (b, 0, 0)),
        out_shape=jax.ShapeDtypeStruct((B, N, D), jnp.float32),
        compiler_params=pltpu.CompilerParams(
            dimension_semantics=("arbitrary",)),
        interpret=interpret,
    )(sc_matrix, W1, b1, W2, b2, W3, b3, g1, be1, g2, be2, g3, be3)

    xf = x.reshape(B, N * D)
    kt = (N * D) // K_TILES
    logits = pl.pallas_call(
        _clf_body,
        grid=(K_TILES,),
        in_specs=[
            pl.BlockSpec((B, kt), lambda k: (0, k)),
            pl.BlockSpec((256, kt), lambda k: (0, k)),
            _full((256,)), _full((256,)), _full((256,)),
            _full((64, 256)), _full((64,)), _full((64,)), _full((64,)),
            _full((2, 64)), _full((2,)),
        ],
        out_specs=pl.BlockSpec((B, 2), lambda k: (0, 0)),
        out_shape=jax.ShapeDtypeStruct((B, 2), jnp.float32),
        scratch_shapes=[pltpu.VMEM((B, 256), jnp.float32)],
        compiler_params=pltpu.CompilerParams(
            dimension_semantics=("arbitrary",)),
        interpret=interpret,
    )(xf, cW1, cb1, lg1, lb1, cW2, cb2, lg2, lb2, cW3, cb3)
    return logits


# speedup vs baseline: 2.1351x; 2.1351x over previous
import functools

import jax
import jax.numpy as jnp
from jax.experimental import pallas as pl
from jax.experimental.pallas import tpu as pltpu

B, N, D = 64, 400, 128
EPS = 1e-5
K_TILES = 8


def _dot_t(x, w):
    return jax.lax.dot_general(
        x.astype(jnp.bfloat16), w.astype(jnp.bfloat16),
        (((1,), (1,)), ((), ())), preferred_element_type=jnp.float32)


def _gcn_body(sc_ref, W1_ref, b1_ref, W2_ref, b2_ref, W3_ref, b3_ref,
              g1_ref, be1_ref, g2_ref, be2_ref, g3_ref, be3_ref, out_ref):
    sc = sc_ref[0]
    row = jax.lax.broadcasted_iota(jnp.int32, (N, N), 0)
    col = jax.lax.broadcasted_iota(jnp.int32, (N, N), 1)
    A = sc + jnp.where((row == col) & (sc == 0.0), 1.0, 0.0)
    deg_c = jnp.sum(A, axis=0, keepdims=True)
    ones = jnp.ones((N, 1), jnp.float32)
    deg_r = jax.lax.dot_general(A, ones, (((0,), (0,)), ((), ())),
                                preferred_element_type=jnp.float32,
                                precision=jax.lax.Precision.HIGHEST)
    dinv_c = jnp.where(deg_c > 0, jax.lax.rsqrt(deg_c), 0.0)
    dinv_r = jnp.where(deg_r > 0, jax.lax.rsqrt(deg_r), 0.0)
    Ms = (A * dinv_r * dinv_c).astype(jnp.bfloat16)

    def prop(y):
        return jax.lax.dot_general(Ms, y.astype(jnp.bfloat16),
                                   (((0,), (0,)), ((), ())),
                                   preferred_element_type=jnp.float32)

    def bn(x, g_ref, b_ref):
        mu = jnp.mean(x, axis=0, keepdims=True)
        var = jnp.mean((x - mu) ** 2, axis=0, keepdims=True)
        return (x - mu) * jax.lax.rsqrt(var + EPS) * g_ref[...] + b_ref[...]

    x = jax.nn.relu(bn(prop(_dot_t(sc, W1_ref[...])) + b1_ref[...],
                       g1_ref, be1_ref))
    x = jax.nn.relu(bn(prop(_dot_t(x, W2_ref[...])) + b2_ref[...],
                       g2_ref, be2_ref))
    x = bn(prop(_dot_t(x, W3_ref[...])) + b3_ref[...], g3_ref, be3_ref)
    out_ref[0] = x.astype(jnp.bfloat16)


def _clf_body(xf_ref, cW1_ref, cb1_ref, lg1_ref, lb1_ref,
              cW2_ref, cb2_ref, lg2_ref, lb2_ref, cW3_ref, cb3_ref,
              out_ref, acc_ref):
    k = pl.program_id(0)

    @pl.when(k == 0)
    def _init():
        acc_ref[...] = jnp.zeros_like(acc_ref)

    acc_ref[...] += jax.lax.dot_general(
        xf_ref[...], cW1_ref[...].astype(jnp.bfloat16),
        (((1,), (1,)), ((), ())), preferred_element_type=jnp.float32)

    @pl.when(k == pl.num_programs(0) - 1)
    def _finish():
        def ln(x, g_ref, b_ref):
            mu = jnp.mean(x, axis=-1, keepdims=True)
            var = jnp.mean((x - mu) ** 2, axis=-1, keepdims=True)
            return (x - mu) * jax.lax.rsqrt(var + EPS) * g_ref[...] + b_ref[...]

        h = acc_ref[...] + cb1_ref[...]
        h = jax.nn.relu(ln(h, lg1_ref, lb1_ref))
        h = _dot_t(h, cW2_ref[...]) + cb2_ref[...]
        h = jax.nn.relu(ln(h, lg2_ref, lb2_ref))
        out_ref[...] = _dot_t(h, cW3_ref[...]) + cb3_ref[...]


def _full(spec_shape):
    nd = len(spec_shape)
    return pl.BlockSpec(spec_shape, lambda *_: (0,) * nd)


@functools.partial(jax.jit, static_argnames=("interpret",))
def kernel(fc_matrix, sc_matrix, W1, b1, W2, b2, W3, b3,
           g1, be1, g2, be2, g3, be3,
           cW1, cb1, lg1, lb1, cW2, cb2, lg2, lb2, cW3, cb3,
           interpret=False):
    del fc_matrix
    x = pl.pallas_call(
        _gcn_body,
        grid=(B,),
        in_specs=[
            pl.BlockSpec((1, N, N), lambda b: (b, 0, 0)),
            _full((D, N)), _full((D,)),
            _full((D, D)), _full((D,)),
            _full((D, D)), _full((D,)),
            _full((D,)), _full((D,)),
            _full((D,)), _full((D,)),
            _full((D,)), _full((D,)),
        ],
        out_specs=pl.BlockSpec((1, N, D), lambda b: (b, 0, 0)),
        out_shape=jax.ShapeDtypeStruct((B, N, D), jnp.bfloat16),
        compiler_params=pltpu.CompilerParams(
            dimension_semantics=("arbitrary",)),
        interpret=interpret,
    )(sc_matrix, W1, b1, W2, b2, W3, b3, g1, be1, g2, be2, g3, be3)

    xf = x.reshape(B, N * D)
    kt = (N * D) // K_TILES
    logits = pl.pallas_call(
        _clf_body,
        grid=(K_TILES,),
        in_specs=[
            pl.BlockSpec((B, kt), lambda k: (0, k)),
            pl.BlockSpec((256, kt), lambda k: (0, k)),
            _full((256,)), _full((256,)), _full((256,)),
            _full((64, 256)), _full((64,)), _full((64,)), _full((64,)),
            _full((2, 64)), _full((2,)),
        ],
        out_specs=pl.BlockSpec((B, 2), lambda k: (0, 0)),
        out_shape=jax.ShapeDtypeStruct((B, 2), jnp.float32),
        scratch_shapes=[pltpu.VMEM((B, 256), jnp.float32)],
        compiler_params=pltpu.CompilerParams(
            dimension_semantics=("arbitrary",)),
        interpret=interpret,
    )(xf, cW1, cb1, lg1, lb1, cW2, cb2, lg2, lb2, cW3, cb3)
    return logits


# speedup vs baseline: 3.0213x; 1.4150x over previous
import functools

import jax
import jax.numpy as jnp
from jax.experimental import pallas as pl
from jax.experimental.pallas import tpu as pltpu

B, N, D = 64, 400, 128
EPS = 1e-5
K_TILES = 8


def _dot_t(x, w):
    return jax.lax.dot_general(
        x.astype(jnp.bfloat16), w.astype(jnp.bfloat16),
        (((1,), (1,)), ((), ())), preferred_element_type=jnp.float32)


def _gcn_body(sc_ref, W1_ref, b1_ref, W2_ref, b2_ref, W3_ref, b3_ref,
              g1_ref, be1_ref, g2_ref, be2_ref, g3_ref, be3_ref, out_ref):
    sc = sc_ref[0]
    row = jax.lax.broadcasted_iota(jnp.int32, (N, N), 0)
    col = jax.lax.broadcasted_iota(jnp.int32, (N, N), 1)
    A = sc + jnp.where((row == col) & (sc == 0.0), 1.0, 0.0)
    deg_c = jnp.sum(A, axis=0, keepdims=True)
    dinv_c = jnp.where(deg_c > 0, jax.lax.rsqrt(deg_c), 0.0)
    dinv_r = jnp.transpose(dinv_c)
    Ms = (A * dinv_r * dinv_c).astype(jnp.bfloat16)

    def prop(y):
        return jax.lax.dot_general(Ms, y.astype(jnp.bfloat16),
                                   (((0,), (0,)), ((), ())),
                                   preferred_element_type=jnp.float32)

    def bn(x, g_ref, b_ref):
        mu = jnp.mean(x, axis=0, keepdims=True)
        var = jnp.mean((x - mu) ** 2, axis=0, keepdims=True)
        return (x - mu) * jax.lax.rsqrt(var + EPS) * g_ref[...] + b_ref[...]

    x = jax.nn.relu(bn(prop(_dot_t(sc, W1_ref[...])) + b1_ref[...],
                       g1_ref, be1_ref))
    x = jax.nn.relu(bn(prop(_dot_t(x, W2_ref[...])) + b2_ref[...],
                       g2_ref, be2_ref))
    x = bn(prop(_dot_t(x, W3_ref[...])) + b3_ref[...], g3_ref, be3_ref)
    out_ref[0] = x.astype(jnp.bfloat16)


def _clf_body(xf_ref, cW1_ref, cb1_ref, lg1_ref, lb1_ref,
              cW2_ref, cb2_ref, lg2_ref, lb2_ref, cW3_ref, cb3_ref,
              out_ref, acc_ref):
    k = pl.program_id(0)

    @pl.when(k == 0)
    def _init():
        acc_ref[...] = jnp.zeros_like(acc_ref)

    acc_ref[...] += jax.lax.dot_general(
        xf_ref[...], cW1_ref[...].astype(jnp.bfloat16),
        (((1,), (1,)), ((), ())), preferred_element_type=jnp.float32)

    @pl.when(k == pl.num_programs(0) - 1)
    def _finish():
        def ln(x, g_ref, b_ref):
            mu = jnp.mean(x, axis=-1, keepdims=True)
            var = jnp.mean((x - mu) ** 2, axis=-1, keepdims=True)
            return (x - mu) * jax.lax.rsqrt(var + EPS) * g_ref[...] + b_ref[...]

        h = acc_ref[...] + cb1_ref[...]
        h = jax.nn.relu(ln(h, lg1_ref, lb1_ref))
        h = _dot_t(h, cW2_ref[...]) + cb2_ref[...]
        h = jax.nn.relu(ln(h, lg2_ref, lb2_ref))
        out_ref[...] = _dot_t(h, cW3_ref[...]) + cb3_ref[...]


def _full(spec_shape):
    nd = len(spec_shape)
    return pl.BlockSpec(spec_shape, lambda *_: (0,) * nd)


@functools.partial(jax.jit, static_argnames=("interpret",))
def kernel(fc_matrix, sc_matrix, W1, b1, W2, b2, W3, b3,
           g1, be1, g2, be2, g3, be3,
           cW1, cb1, lg1, lb1, cW2, cb2, lg2, lb2, cW3, cb3,
           interpret=False):
    del fc_matrix
    x = pl.pallas_call(
        _gcn_body,
        grid=(B,),
        in_specs=[
            pl.BlockSpec((1, N, N), lambda b: (b, 0, 0)),
            _full((D, N)), _full((D,)),
            _full((D, D)), _full((D,)),
            _full((D, D)), _full((D,)),
            _full((D,)), _full((D,)),
            _full((D,)), _full((D,)),
            _full((D,)), _full((D,)),
        ],
        out_specs=pl.BlockSpec((1, N, D), lambda b: (b, 0, 0)),
        out_shape=jax.ShapeDtypeStruct((B, N, D), jnp.bfloat16),
        compiler_params=pltpu.CompilerParams(
            dimension_semantics=("arbitrary",)),
        interpret=interpret,
    )(sc_matrix, W1, b1, W2, b2, W3, b3, g1, be1, g2, be2, g3, be3)

    xf = x.reshape(B, N * D)
    kt = (N * D) // K_TILES
    logits = pl.pallas_call(
        _clf_body,
        grid=(K_TILES,),
        in_specs=[
            pl.BlockSpec((B, kt), lambda k: (0, k)),
            pl.BlockSpec((256, kt), lambda k: (0, k)),
            _full((256,)), _full((256,)), _full((256,)),
            _full((64, 256)), _full((64,)), _full((64,)), _full((64,)),
            _full((2, 64)), _full((2,)),
        ],
        out_specs=pl.BlockSpec((B, 2), lambda k: (0, 0)),
        out_shape=jax.ShapeDtypeStruct((B, 2), jnp.float32),
        scratch_shapes=[pltpu.VMEM((B, 256), jnp.float32)],
        compiler_params=pltpu.CompilerParams(
            dimension_semantics=("arbitrary",)),
        interpret=interpret,
    )(xf, cW1, cb1, lg1, lb1, cW2, cb2, lg2, lb2, cW3, cb3)
    return logits


# speedup vs baseline: 7.5549x; 2.5006x over previous
import functools

import jax
import jax.numpy as jnp
from jax.experimental import pallas as pl
from jax.experimental.pallas import tpu as pltpu

B, N, D = 64, 400, 128
EPS = 1e-5
GB = 8
GSTEPS = B // GB
K_TILES = 5
NT = N // K_TILES


def _dot_t(x, w):
    return jax.lax.dot_general(
        x.astype(jnp.bfloat16), w.astype(jnp.bfloat16),
        (((1,), (1,)), ((), ())), preferred_element_type=jnp.float32)


def _body(sc_ref, W1_ref, b1_ref, W2_ref, b2_ref, W3_ref, b3_ref,
          g1_ref, be1_ref, g2_ref, be2_ref, g3_ref, be3_ref,
          cW1_ref, cb1_ref, lg1_ref, lb1_ref,
          cW2_ref, cb2_ref, lg2_ref, lb2_ref, cW3_ref, cb3_ref,
          out_ref, xs_ref, acc_ref):
    i = pl.program_id(0)

    @pl.when(i < GSTEPS)
    def _gcn():
        sc = sc_ref[...]
        row = jax.lax.broadcasted_iota(jnp.int32, (GB, N, N), 1)
        col = jax.lax.broadcasted_iota(jnp.int32, (GB, N, N), 2)
        A = sc + jnp.where((row == col) & (sc == 0.0), 1.0, 0.0)
        deg_c = jnp.sum(A, axis=1, keepdims=True)
        dinv_c = jnp.where(deg_c > 0, jax.lax.rsqrt(deg_c), 0.0)
        dinv_r = jnp.transpose(dinv_c, (0, 2, 1))
        Ms = (A * dinv_r * dinv_c).astype(jnp.bfloat16)

        def prop(y):
            zs = [jax.lax.dot_general(Ms[g], y[g].astype(jnp.bfloat16),
                                      (((0,), (0,)), ((), ())),
                                      preferred_element_type=jnp.float32)
                  for g in range(GB)]
            return jnp.stack(zs, axis=0)

        def feat(x, w_ref):
            h = _dot_t(x.reshape(GB * N, x.shape[-1]), w_ref[...])
            return h.reshape(GB, N, D)

        def bn(x, g_ref, b_ref):
            mu = jnp.mean(x, axis=1, keepdims=True)
            var = jnp.mean((x - mu) ** 2, axis=1, keepdims=True)
            return (x - mu) * jax.lax.rsqrt(var + EPS) * g_ref[...] + b_ref[...]

        x = jax.nn.relu(bn(prop(feat(sc, W1_ref)) + b1_ref[...],
                           g1_ref, be1_ref))
        x = jax.nn.relu(bn(prop(feat(x, W2_ref)) + b2_ref[...],
                           g2_ref, be2_ref))
        x = bn(prop(feat(x, W3_ref)) + b3_ref[...], g3_ref, be3_ref)
        xs_ref[pl.ds(i * GB, GB)] = x.astype(jnp.bfloat16)

    @pl.when(i == GSTEPS)
    def _init():
        acc_ref[...] = jnp.zeros_like(acc_ref)

    @pl.when(i >= GSTEPS)
    def _clf():
        k = i - GSTEPS
        xt = xs_ref[:, pl.ds(k * NT, NT), :]
        w = cW1_ref[...].astype(jnp.bfloat16)
        parts = [
            jax.lax.dot_general(
                xt[:, j, :], w[:, j * D:(j + 1) * D],
                (((1,), (1,)), ((), ())), preferred_element_type=jnp.float32)
            for j in range(NT)
        ]
        while len(parts) > 1:
            rest = [parts[-1]] if len(parts) % 2 else []
            parts = [a + b for a, b in zip(parts[::2], parts[1::2])] + rest
        acc_ref[...] += parts[0]

    @pl.when(i == GSTEPS + K_TILES - 1)
    def _finish():
        def ln(x, g_ref, b_ref):
            mu = jnp.mean(x, axis=-1, keepdims=True)
            var = jnp.mean((x - mu) ** 2, axis=-1, keepdims=True)
            return (x - mu) * jax.lax.rsqrt(var + EPS) * g_ref[...] + b_ref[...]

        h = acc_ref[...] + cb1_ref[...]
        h = jax.nn.relu(ln(h, lg1_ref, lb1_ref))
        h = _dot_t(h, cW2_ref[...]) + cb2_ref[...]
        h = jax.nn.relu(ln(h, lg2_ref, lb2_ref))
        out_ref[...] = _dot_t(h, cW3_ref[...]) + cb3_ref[...]


def _full(spec_shape):
    nd = len(spec_shape)
    return pl.BlockSpec(spec_shape, lambda *_: (0,) * nd)


@functools.partial(jax.jit, static_argnames=("interpret",))
def kernel(fc_matrix, sc_matrix, W1, b1, W2, b2, W3, b3,
           g1, be1, g2, be2, g3, be3,
           cW1, cb1, lg1, lb1, cW2, cb2, lg2, lb2, cW3, cb3,
           interpret=False):
    del fc_matrix
    kt = NT * D
    logits = pl.pallas_call(
        _body,
        grid=(GSTEPS + K_TILES,),
        in_specs=[
            pl.BlockSpec((GB, N, N),
                         lambda i: (jnp.minimum(i, GSTEPS - 1), 0, 0)),
            _full((D, N)), _full((D,)),
            _full((D, D)), _full((D,)),
            _full((D, D)), _full((D,)),
            _full((D,)), _full((D,)),
            _full((D,)), _full((D,)),
            _full((D,)), _full((D,)),
            pl.BlockSpec((256, kt),
                         lambda i: (0, jnp.maximum(i - GSTEPS, 0))),
            _full((256,)), _full((256,)), _full((256,)),
            _full((64, 256)), _full((64,)), _full((64,)), _full((64,)),
            _full((2, 64)), _full((2,)),
        ],
        out_specs=pl.BlockSpec((B, 2), lambda i: (0, 0)),
        out_shape=jax.ShapeDtypeStruct((B, 2), jnp.float32),
        scratch_shapes=[pltpu.VMEM((B, N, D), jnp.bfloat16),
                        pltpu.VMEM((B, 256), jnp.float32)],
        compiler_params=pltpu.CompilerParams(
            dimension_semantics=("arbitrary",)),
        interpret=interpret,
    )(sc_matrix, W1, b1, W2, b2, W3, b3, g1, be1, g2, be2, g3, be3,
      cW1, cb1, lg1, lb1, cW2, cb2, lg2, lb2, cW3, cb3)
    return logits
